# baseline (speedup 1.0000x reference)
import functools

import jax
import jax.numpy as jnp
from jax import lax
from jax.experimental import pallas as pl
from jax.experimental.pallas import tpu as pltpu
from jax.experimental.pallas import tpu_sc as plsc

N = 10000
E = 320000
NF = 14
DIM = 128
NG = 256

NW = 32
CHUNK = 128
ECH = 80
EP = NW * ECH * CHUNK
NT = 10240
ROWS_PER_TILE = NT // 16

PCH = 64
PNCH = 5
NODES_PER_W = PCH * PNCH
NGT = 384


def _sc_segsum():
    mesh = plsc.VectorSubcoreMesh(core_axis_name="c", subcore_axis_name="s")

    @functools.partial(
        pl.kernel,
        out_type=jax.ShapeDtypeStruct((2 * N, DIM), jnp.float32),
        mesh=mesh,
        scratch_types=[
            pltpu.VMEM((ECH, CHUNK), jnp.int32),
            pltpu.VMEM((2, CHUNK), jnp.int32),
            pltpu.VMEM((2, CHUNK), jnp.int32),
            pltpu.VMEM((CHUNK, DIM), jnp.float32),
            pltpu.VMEM((CHUNK, DIM), jnp.float32),
            pltpu.VMEM_SHARED((NT, DIM), jnp.float32),
            pltpu.SemaphoreType.DMA,
            pltpu.SemaphoreType.DMA,
        ],
    )
    def k(m_hbm, packed_hbm, zeros_hbm, out_hbm,
          packed_v, srcc, dstc, r0, r1, agg_sh, gs0, gs1):
        gsem = (gs0, gs1)
        rows = (r0, r1)
        c = lax.axis_index("c")
        s = lax.axis_index("s")
        w = c * 16 + s

        def unpack(row, slot):
            for i in range(CHUNK // 16):
                v = packed_v[row, pl.ds(16 * i, 16)]
                srcc[slot, pl.ds(16 * i, 16)] = lax.bitwise_and(v, 0xFFFF)
                dstc[slot, pl.ds(16 * i, 16)] = lax.shift_right_logical(v, 16)

        pltpu.sync_copy(packed_hbm.at[w], packed_v)
        unpack(0, 0)
        pltpu.async_copy(m_hbm.at[srcc.at[0]], r0, gsem[0])

        for z in range(ROWS_PER_TILE // CHUNK):
            pltpu.sync_copy(
                zeros_hbm, agg_sh.at[pl.ds(s * ROWS_PER_TILE + z * CHUNK, CHUNK)])
        plsc.subcore_barrier()

        def wait_gather(b):
            pltpu.make_async_copy(m_hbm.at[pl.ds(0, CHUNK)], rows[b], gsem[b]).wait()

        def body(p, carry):
            for b in range(2):
                cc = 2 * p + b
                nb = 1 - b
                wait_gather(b)
                nxt = jnp.minimum(cc + 1, ECH - 1)
                unpack(nxt, nb)
                pltpu.async_copy(m_hbm.at[srcc.at[nb]], rows[nb], gsem[nb])
                pltpu.sync_copy(rows[b], agg_sh.at[dstc.at[b]], add=True)
            return carry

        lax.fori_loop(0, ECH // 2, body, 0, unroll=False)
        wait_gather(0)
        plsc.subcore_barrier()

        @pl.when(s < 15)
        def _():
            pltpu.sync_copy(
                agg_sh.at[pl.ds(s * ROWS_PER_TILE, ROWS_PER_TILE)],
                out_hbm.at[pl.ds(c * N + s * ROWS_PER_TILE, ROWS_PER_TILE)],
            )

        @pl.when(s == 15)
        def _():
            pltpu.sync_copy(
                agg_sh.at[pl.ds(15 * ROWS_PER_TILE, N - 15 * ROWS_PER_TILE)],
                out_hbm.at[pl.ds(c * N + 15 * ROWS_PER_TILE,
                                 N - 15 * ROWS_PER_TILE)],
            )

    return k


def _sc_pool():
    mesh = plsc.VectorSubcoreMesh(core_axis_name="c", subcore_axis_name="s")

    @functools.partial(
        pl.kernel,
        out_type=jax.ShapeDtypeStruct((2 * NG, DIM), jnp.float32),
        mesh=mesh,
        scratch_types=[
            pltpu.VMEM((PNCH, PCH), jnp.int32),
            pltpu.VMEM((PCH, DIM), jnp.float32),
            pltpu.VMEM_SHARED((NGT, DIM), jnp.float32),
        ],
    )
    def k(h_hbm, batchp_hbm, zeros_hbm, out_hbm, bidx_v, rows_v, pool_sh):
        c = lax.axis_index("c")
        s = lax.axis_index("s")
        w = c * 16 + s
        pltpu.sync_copy(zeros_hbm.at[pl.ds(0, NGT // 16)],
                        pool_sh.at[pl.ds(s * (NGT // 16), NGT // 16)])
        pltpu.sync_copy(batchp_hbm.at[w], bidx_v)
        plsc.subcore_barrier()

        def body(j, _):
            pltpu.sync_copy(h_hbm.at[pl.ds(w * NODES_PER_W + j * PCH, PCH)], rows_v)
            pltpu.sync_copy(rows_v, pool_sh.at[bidx_v.at[j]], add=True)
            return _

        lax.fori_loop(0, PNCH, body, 0, unroll=False)
        plsc.subcore_barrier()
        pltpu.sync_copy(
            pool_sh.at[pl.ds(s * (NG // 16), NG // 16)],
            out_hbm.at[pl.ds(c * NG + s * (NG // 16), NG // 16)],
        )

    return k


RB = 1000
NBLK = N // RB


def _tc_first(x_ref, w1r_ref, w1o_ref, b1_ref, m_ref, r_ref):
    x = x_ref[...]
    m_ref[...] = lax.dot_general(x, w1r_ref[...], (((1,), (1,)), ((), ())),
                                 preferred_element_type=jnp.float32)
    r_ref[...] = lax.dot_general(x, w1o_ref[...], (((1,), (1,)), ((), ())),
                                 preferred_element_type=jnp.float32) + b1_ref[...]


def _tc_mid(a0_ref, a1_ref, root_ref, wr_ref, wo_ref, b_ref, m_ref, r_ref):
    h = jnp.maximum(a0_ref[...] + a1_ref[...] + root_ref[...], 0.0)
    m_ref[...] = lax.dot_general(h, wr_ref[...], (((1,), (1,)), ((), ())),
                                 preferred_element_type=jnp.float32)
    r_ref[...] = lax.dot_general(h, wo_ref[...], (((1,), (1,)), ((), ())),
                                 preferred_element_type=jnp.float32) + b_ref[...]


def _tc_last(a0_ref, a1_ref, root_ref, h_ref):
    h_ref[...] = jnp.maximum(a0_ref[...] + a1_ref[...] + root_ref[...], 0.0)


def _tc_head(p0_ref, p1_ref, w1_ref, b1_ref, w2_ref, b2_ref, out_ref):
    p = p0_ref[...] + p1_ref[...]
    h2 = jnp.maximum(
        lax.dot_general(p, w1_ref[...], (((1,), (1,)), ((), ())),
                        preferred_element_type=jnp.float32) + b1_ref[...], 0.0)
    lg = lax.dot_general(h2, w2_ref[...], (((1,), (1,)), ((), ())),
                         preferred_element_type=jnp.float32) + b2_ref[...]
    col = lax.broadcasted_iota(jnp.int32, lg.shape, 1)
    lg = jnp.where(col < 2, lg, -1e30)
    mx = jnp.max(lg, axis=1, keepdims=True)
    lse = mx + jnp.log(jnp.sum(jnp.exp(lg - mx), axis=1, keepdims=True))
    out_ref[...] = lg - lse


def kernel(x, edge_index, batch, W1r, b1r, W1o, W2r, b2r, W2o, W3r, b3r, W3o,
           W4r, b4r, W4o, W5r, b5r, W5o, lin1_W, lin1_b, lin2_W, lin2_b):
    f32 = jnp.float32
    src = edge_index[0]
    dst = edge_index[1]
    pad = EP - E
    srcp = jnp.concatenate([src, jnp.zeros((pad,), jnp.int32)])
    dstp = jnp.concatenate([dst, jnp.full((pad,), N, jnp.int32)])
    packed = jnp.bitwise_or(srcp, dstp << 16).reshape(NW, ECH, CHUNK)
    zeros_blk = jnp.zeros((CHUNK, DIM), f32)

    batchp = jnp.concatenate([batch, jnp.full((NT - N,), NG, jnp.int32)]
                             ).reshape(NW, PNCH, PCH)

    sc_seg = _sc_segsum()
    sc_pool = _sc_pool()

    row_blk = pl.BlockSpec((RB, DIM), lambda i: (i, 0))
    a1_blk = pl.BlockSpec((RB, DIM), lambda i: (i + NBLK, 0))
    full_w = pl.BlockSpec((DIM, DIM), lambda i: (0, 0))
    full_b = pl.BlockSpec((1, DIM), lambda i: (0, 0))

    first = pl.pallas_call(
        _tc_first,
        grid=(NBLK,),
        in_specs=[pl.BlockSpec((RB, NF), lambda i: (i, 0)),
                  pl.BlockSpec((DIM, NF), lambda i: (0, 0)),
                  pl.BlockSpec((DIM, NF), lambda i: (0, 0)),
                  full_b],
        out_specs=[row_blk, row_blk],
        out_shape=[jax.ShapeDtypeStruct((N, DIM), f32)] * 2,
    )
    mid = pl.pallas_call(
        _tc_mid,
        grid=(NBLK,),
        in_specs=[row_blk, a1_blk, row_blk, full_w, full_w, full_b],
        out_specs=[row_blk, row_blk],
        out_shape=[jax.ShapeDtypeStruct((N, DIM), f32)] * 2,
    )
    last = pl.pallas_call(
        _tc_last,
        grid=(NBLK,),
        in_specs=[row_blk, a1_blk, row_blk],
        out_specs=row_blk,
        out_shape=jax.ShapeDtypeStruct((NT, DIM), f32),
    )
    head = pl.pallas_call(
        _tc_head,
        grid=(1,),
        in_specs=[pl.BlockSpec((NG, DIM), lambda i: (0, 0)),
                  pl.BlockSpec((NG, DIM), lambda i: (1, 0)),
                  pl.BlockSpec((DIM, DIM), lambda i: (0, 0)),
                  pl.BlockSpec((1, DIM), lambda i: (0, 0)),
                  pl.BlockSpec((DIM, DIM), lambda i: (0, 0)),
                  pl.BlockSpec((1, DIM), lambda i: (0, 0))],
        out_specs=pl.BlockSpec((NG, DIM), lambda i: (0, 0)),
        out_shape=jax.ShapeDtypeStruct((NG, DIM), f32),
    )

    m, root = first(x, W1r, W1o, b1r.reshape(1, DIM))
    Wrs = [W2r, W3r, W4r, W5r]
    Wos = [W2o, W3o, W4o, W5o]
    bs = [b2r, b3r, b4r, b5r]
    for i in range(4):
        agg = sc_seg(m, packed, zeros_blk)
        m, root = mid(agg, agg, root, Wrs[i], Wos[i], bs[i].reshape(1, DIM))
    agg = sc_seg(m, packed, zeros_blk)
    h5 = last(agg, agg, root)
    pooled = sc_pool(h5, batchp, zeros_blk)

    lin2_Wp = jnp.concatenate([lin2_W, jnp.zeros((DIM - 2, DIM), f32)], axis=0)
    lin2_bp = jnp.concatenate([lin2_b, jnp.zeros((DIM - 2,), f32)]).reshape(1, DIM)
    out = head(pooled, pooled, lin1_W, lin1_b.reshape(1, DIM), lin2_Wp, lin2_bp)
    return out[:, :2]


# speedup vs baseline: 1.6759x; 1.6759x over previous
import functools

import jax
import jax.numpy as jnp
from jax import lax
from jax.experimental import pallas as pl
from jax.experimental.pallas import tpu as pltpu
from jax.experimental.pallas import tpu_sc as plsc

N = 10000
E = 320000
NF = 14
DIM = 128
NG = 256

NW = 32
CHUNK = 128
ECHF = 120
ECHS = 38
FAST_C = 0
NT = 10112
ROWS_PER_TILE = NT // 16
HP = 10240

PCH = 64
PNCH = 5
NODES_PER_W = PCH * PNCH
NGT = 384


def _sc_segsum():
    mesh = plsc.VectorSubcoreMesh(core_axis_name="c", subcore_axis_name="s")

    @functools.partial(
        pl.kernel,
        out_type=jax.ShapeDtypeStruct((2 * N, DIM), jnp.float32),
        mesh=mesh,
        scratch_types=[
            pltpu.VMEM((ECHF, CHUNK), jnp.int32),
            pltpu.VMEM((2, CHUNK), jnp.int32),
            pltpu.VMEM((2, CHUNK), jnp.int32),
            pltpu.VMEM((CHUNK, DIM), jnp.float32),
            pltpu.VMEM((CHUNK, DIM), jnp.float32),
            pltpu.VMEM_SHARED((NT, DIM), jnp.float32),
            pltpu.SemaphoreType.DMA,
            pltpu.SemaphoreType.DMA,
        ],
    )
    def k(m_hbm, packed_hbm, zeros_hbm, out_hbm,
          packed_v, srcc, dstc, r0, r1, agg_sh, gs0, gs1):
        gsem = (gs0, gs1)
        rows = (r0, r1)
        c = lax.axis_index("c")
        s = lax.axis_index("s")
        w = c * 16 + s
        nch = jnp.where(c == FAST_C, ECHF, ECHS)

        def unpack(row, slot):
            for i in range(CHUNK // 16):
                v = packed_v[row, pl.ds(16 * i, 16)]
                srcc[slot, pl.ds(16 * i, 16)] = lax.bitwise_and(v, 0xFFFF)
                dstc[slot, pl.ds(16 * i, 16)] = lax.shift_right_logical(v, 16)

        pltpu.sync_copy(packed_hbm.at[w], packed_v)
        unpack(0, 0)
        pltpu.async_copy(m_hbm.at[srcc.at[0]], r0, gsem[0])

        for z in range(4):
            pltpu.sync_copy(
                zeros_hbm, agg_sh.at[pl.ds(s * ROWS_PER_TILE + z * CHUNK, CHUNK)])
        pltpu.sync_copy(
            zeros_hbm.at[pl.ds(0, ROWS_PER_TILE - 4 * CHUNK)],
            agg_sh.at[pl.ds(s * ROWS_PER_TILE + 4 * CHUNK,
                            ROWS_PER_TILE - 4 * CHUNK)])
        plsc.subcore_barrier()

        def wait_gather(b):
            pltpu.make_async_copy(m_hbm.at[pl.ds(0, CHUNK)], rows[b], gsem[b]).wait()

        def body(p, carry):
            for b in range(2):
                cc = 2 * p + b
                nb = 1 - b
                wait_gather(b)
                nxt = jnp.minimum(cc + 1, nch - 1)
                unpack(nxt, nb)
                pltpu.async_copy(m_hbm.at[srcc.at[nb]], rows[nb], gsem[nb])
                pltpu.sync_copy(rows[b], agg_sh.at[dstc.at[b]], add=True)
            return carry

        lax.fori_loop(0, nch // 2, body, 0, unroll=False)
        wait_gather(0)
        plsc.subcore_barrier()

        @pl.when(s < 15)
        def _():
            pltpu.sync_copy(
                agg_sh.at[pl.ds(s * ROWS_PER_TILE, ROWS_PER_TILE)],
                out_hbm.at[pl.ds(c * N + s * ROWS_PER_TILE, ROWS_PER_TILE)],
            )

        @pl.when(s == 15)
        def _():
            pltpu.sync_copy(
                agg_sh.at[pl.ds(15 * ROWS_PER_TILE, N - 15 * ROWS_PER_TILE)],
                out_hbm.at[pl.ds(c * N + 15 * ROWS_PER_TILE,
                                 N - 15 * ROWS_PER_TILE)],
            )

    return k


def _sc_pool():
    mesh = plsc.VectorSubcoreMesh(core_axis_name="c", subcore_axis_name="s")

    @functools.partial(
        pl.kernel,
        out_type=jax.ShapeDtypeStruct((2 * NG, DIM), jnp.float32),
        mesh=mesh,
        scratch_types=[
            pltpu.VMEM((PNCH, PCH), jnp.int32),
            pltpu.VMEM((PCH, DIM), jnp.float32),
            pltpu.VMEM_SHARED((NGT, DIM), jnp.float32),
        ],
    )
    def k(h_hbm, batchp_hbm, zeros_hbm, out_hbm, bidx_v, rows_v, pool_sh):
        c = lax.axis_index("c")
        s = lax.axis_index("s")
        w = c * 16 + s
        pltpu.sync_copy(zeros_hbm.at[pl.ds(0, NGT // 16)],
                        pool_sh.at[pl.ds(s * (NGT // 16), NGT // 16)])
        pltpu.sync_copy(batchp_hbm.at[w], bidx_v)
        plsc.subcore_barrier()

        def body(j, _):
            pltpu.sync_copy(h_hbm.at[pl.ds(w * NODES_PER_W + j * PCH, PCH)], rows_v)
            pltpu.sync_copy(rows_v, pool_sh.at[bidx_v.at[j]], add=True)
            return _

        lax.fori_loop(0, PNCH, body, 0, unroll=False)
        plsc.subcore_barrier()
        pltpu.sync_copy(
            pool_sh.at[pl.ds(s * (NG // 16), NG // 16)],
            out_hbm.at[pl.ds(c * NG + s * (NG // 16), NG // 16)],
        )

    return k


RB = 1000
NBLK = N // RB


def _tc_first(x_ref, w1r_ref, w1o_ref, b1_ref, m_ref, r_ref):
    x = x_ref[...]
    m_ref[...] = lax.dot_general(x, w1r_ref[...], (((1,), (1,)), ((), ())),
                                 preferred_element_type=jnp.float32)
    r_ref[...] = lax.dot_general(x, w1o_ref[...], (((1,), (1,)), ((), ())),
                                 preferred_element_type=jnp.float32) + b1_ref[...]


def _tc_mid(a0_ref, a1_ref, root_ref, wr_ref, wo_ref, b_ref, m_ref, r_ref):
    h = jnp.maximum(a0_ref[...] + a1_ref[...] + root_ref[...], 0.0)
    m_ref[...] = lax.dot_general(h, wr_ref[...], (((1,), (1,)), ((), ())),
                                 preferred_element_type=jnp.float32)
    r_ref[...] = lax.dot_general(h, wo_ref[...], (((1,), (1,)), ((), ())),
                                 preferred_element_type=jnp.float32) + b_ref[...]


def _tc_last(a0_ref, a1_ref, root_ref, h_ref):
    h_ref[...] = jnp.maximum(a0_ref[...] + a1_ref[...] + root_ref[...], 0.0)


def _tc_head(p0_ref, p1_ref, w1_ref, b1_ref, w2_ref, b2_ref, out_ref):
    p = p0_ref[...] + p1_ref[...]
    h2 = jnp.maximum(
        lax.dot_general(p, w1_ref[...], (((1,), (1,)), ((), ())),
                        preferred_element_type=jnp.float32) + b1_ref[...], 0.0)
    lg = lax.dot_general(h2, w2_ref[...], (((1,), (1,)), ((), ())),
                         preferred_element_type=jnp.float32) + b2_ref[...]
    col = lax.broadcasted_iota(jnp.int32, lg.shape, 1)
    lg = jnp.where(col < 2, lg, -1e30)
    mx = jnp.max(lg, axis=1, keepdims=True)
    lse = mx + jnp.log(jnp.sum(jnp.exp(lg - mx), axis=1, keepdims=True))
    out_ref[...] = lg - lse


def kernel(x, edge_index, batch, W1r, b1r, W1o, W2r, b2r, W2o, W3r, b3r, W3o,
           W4r, b4r, W4o, W5r, b5r, W5o, lin1_W, lin1_b, lin2_W, lin2_b):
    f32 = jnp.float32
    src = edge_index[0]
    dst = edge_index[1]
    nfast = 16 * ECHF * CHUNK
    nslow_used = E - nfast
    pad = 16 * ECHS * CHUNK - nslow_used
    pk = jnp.bitwise_or(src, dst << 16)
    dump_pk = jnp.full((pad,), N << 16, jnp.int32)
    fast_blk = pk[:nfast].reshape(16, ECHF, CHUNK)
    slow_blk = jnp.concatenate([pk[nfast:], dump_pk]).reshape(16, ECHS, CHUNK)
    slow_blk = jnp.pad(slow_blk, ((0, 0), (0, ECHF - ECHS), (0, 0)),
                       constant_values=N << 16)
    blocks = [fast_blk, slow_blk] if FAST_C == 0 else [slow_blk, fast_blk]
    packed = jnp.concatenate(blocks)
    zeros_blk = jnp.zeros((CHUNK, DIM), f32)

    batchp = jnp.concatenate([batch, jnp.full((HP - N,), NG, jnp.int32)]
                             ).reshape(NW, PNCH, PCH)

    sc_seg = _sc_segsum()
    sc_pool = _sc_pool()

    row_blk = pl.BlockSpec((RB, DIM), lambda i: (i, 0))
    a1_blk = pl.BlockSpec((RB, DIM), lambda i: (i + NBLK, 0))
    full_w = pl.BlockSpec((DIM, DIM), lambda i: (0, 0))
    full_b = pl.BlockSpec((1, DIM), lambda i: (0, 0))

    first = pl.pallas_call(
        _tc_first,
        grid=(NBLK,),
        in_specs=[pl.BlockSpec((RB, NF), lambda i: (i, 0)),
                  pl.BlockSpec((DIM, NF), lambda i: (0, 0)),
                  pl.BlockSpec((DIM, NF), lambda i: (0, 0)),
                  full_b],
        out_specs=[row_blk, row_blk],
        out_shape=[jax.ShapeDtypeStruct((N, DIM), f32)] * 2,
    )
    mid = pl.pallas_call(
        _tc_mid,
        grid=(NBLK,),
        in_specs=[row_blk, a1_blk, row_blk, full_w, full_w, full_b],
        out_specs=[row_blk, row_blk],
        out_shape=[jax.ShapeDtypeStruct((N, DIM), f32)] * 2,
    )
    last = pl.pallas_call(
        _tc_last,
        grid=(NBLK,),
        in_specs=[row_blk, a1_blk, row_blk],
        out_specs=row_blk,
        out_shape=jax.ShapeDtypeStruct((HP, DIM), f32),
    )
    head = pl.pallas_call(
        _tc_head,
        grid=(1,),
        in_specs=[pl.BlockSpec((NG, DIM), lambda i: (0, 0)),
                  pl.BlockSpec((NG, DIM), lambda i: (1, 0)),
                  pl.BlockSpec((DIM, DIM), lambda i: (0, 0)),
                  pl.BlockSpec((1, DIM), lambda i: (0, 0)),
                  pl.BlockSpec((DIM, DIM), lambda i: (0, 0)),
                  pl.BlockSpec((1, DIM), lambda i: (0, 0))],
        out_specs=pl.BlockSpec((NG, DIM), lambda i: (0, 0)),
        out_shape=jax.ShapeDtypeStruct((NG, DIM), f32),
    )

    m, root = first(x, W1r, W1o, b1r.reshape(1, DIM))
    Wrs = [W2r, W3r, W4r, W5r]
    Wos = [W2o, W3o, W4o, W5o]
    bs = [b2r, b3r, b4r, b5r]
    for i in range(4):
        agg = sc_seg(m, packed, zeros_blk)
        m, root = mid(agg, agg, root, Wrs[i], Wos[i], bs[i].reshape(1, DIM))
    agg = sc_seg(m, packed, zeros_blk)
    h5 = last(agg, agg, root)
    pooled = sc_pool(h5, batchp, zeros_blk)

    lin2_Wp = jnp.concatenate([lin2_W, jnp.zeros((DIM - 2, DIM), f32)], axis=0)
    lin2_bp = jnp.concatenate([lin2_b, jnp.zeros((DIM - 2,), f32)]).reshape(1, DIM)
    out = head(pooled, pooled, lin1_W, lin1_b.reshape(1, DIM), lin2_Wp, lin2_bp)
    return out[:, :2]
